# baseline (speedup 1.0000x reference)
import functools

import jax
import jax.numpy as jnp
from jax import lax
from jax.experimental import pallas as pl
from jax.experimental.pallas import tpu as pltpu
from jax.experimental.pallas import tpu_sc as plsc


_B_BLK = 4


def _argmin_body(xt_ref, et_ref, ind3_ref, e2c_ref, et2_ref):
    t = xt_ref.shape[2]
    k = et_ref.shape[1]

    @pl.when(pl.program_id(0) == 0)
    def _():
        et = et_ref[...]
        e2 = jnp.sum(et * et, axis=0, keepdims=True)
        e2c_ref[...] = jnp.transpose(e2)
        et2_ref[...] = et + et

    e2c = e2c_ref[...]
    et2 = et2_ref[...]
    iota_f = lax.broadcasted_iota(jnp.int32, (k, t), 0).astype(jnp.float32)
    for b in range(_B_BLK):
        xb = xt_ref[b]
        x2 = jnp.sum(xb * xb, axis=0, keepdims=True)
        xe2 = lax.dot_general(
            et2, xb, (((0,), (0,)), ((), ())),
            preferred_element_type=jnp.float32)
        pre = (x2 - xe2) + e2c
        m = jnp.min(pre, axis=0, keepdims=True)
        indf = jnp.min(jnp.where(pre == m, iota_f, float(k)),
                       axis=0, keepdims=True)
        ind3_ref[b] = indf.astype(jnp.int32)


def _argmin_indices(x_t, embed_t):
    b, d, t = x_t.shape
    k = embed_t.shape[1]
    ind3 = pl.pallas_call(
        _argmin_body,
        grid=(b // _B_BLK,),
        in_specs=[
            pl.BlockSpec((_B_BLK, d, t), lambda i: (i, 0, 0)),
            pl.BlockSpec((d, k), lambda i: (0, 0)),
        ],
        out_specs=pl.BlockSpec((_B_BLK, 1, t), lambda i: (i, 0, 0)),
        out_shape=jax.ShapeDtypeStruct((b, 1, t), jnp.int32),
        scratch_shapes=[
            pltpu.VMEM((k, 1), jnp.float32),
            pltpu.VMEM((d, k), jnp.float32),
        ],
    )(x_t, embed_t)
    return ind3


@functools.cache
def _make_sc_gather(K, D, B, T):
    info = plsc.get_sparse_core_info()
    NC, NS, L = info.num_cores, info.num_subcores, info.num_lanes
    NW = NC * NS
    BG, DG = 8, 4
    BPW, DPW = B // BG, D // DG
    assert BG * DG == NW and T % L == 0
    mesh = plsc.VectorSubcoreMesh(core_axis_name="c", subcore_axis_name="s")

    @functools.partial(
        pl.kernel,
        mesh=mesh,
        out_type=jax.ShapeDtypeStruct((B, D, T), jnp.float32),
        scratch_types=[
            pltpu.VMEM((BPW * T,), jnp.int32),
            pltpu.VMEM((K * DPW,), jnp.float32),
            pltpu.VMEM((BPW, DPW, T), jnp.float32),
        ],
        compiler_params=pltpu.CompilerParams(needs_layout_passes=False),
    )
    def gather(et_flat_hbm, idx_hbm, out_hbm, idx_v, tab_v, out_v):
        wid = lax.axis_index("s") * NC + lax.axis_index("c")
        bg = wid // DG
        dg = wid % DG
        pltpu.sync_copy(idx_hbm.at[pl.ds(bg * (BPW * T), BPW * T)], idx_v)
        pltpu.sync_copy(et_flat_hbm.at[pl.ds(dg * (DPW * K), DPW * K)], tab_v)

        for b in range(BPW):
            def body(g, _, b=b):
                col = g * L
                idx16 = idx_v[pl.ds(b * T + col, L)]
                vals = [plsc.load_gather(tab_v, [idx16 + (d * K)])
                        for d in range(DPW)]
                for d in range(DPW):
                    out_v[b, d, pl.ds(col, L)] = vals[d]
                return _

            lax.fori_loop(0, T // L, body, None, unroll=4)

        pltpu.sync_copy(
            out_v, out_hbm.at[pl.ds(bg * BPW, BPW), pl.ds(dg * DPW, DPW)])

    return gather


def kernel(x, embed):
    b, t, d = x.shape
    k = embed.shape[0]
    x_t = x.transpose(0, 2, 1)
    embed_t = embed.T
    ind3 = _argmin_indices(x_t, embed_t)
    ind1 = ind3.reshape(b * t)
    quant_t = _make_sc_gather(k, d, b, t)(embed_t.reshape(k * d), ind1)
    return quant_t.transpose(0, 2, 1), ind3.reshape(b, t)


# speedup vs baseline: 1.0171x; 1.0171x over previous
import functools

import jax
import jax.numpy as jnp
from jax import lax
from jax.experimental import pallas as pl
from jax.experimental.pallas import tpu as pltpu
from jax.experimental.pallas import tpu_sc as plsc


_B_BLK = 4


def _argmin_body(xt_ref, et_ref, ind3_ref, e2c_ref, et2_ref):
    t = xt_ref.shape[2]
    k = et_ref.shape[1]

    @pl.when(pl.program_id(0) == 0)
    def _():
        et = et_ref[...]
        e2 = jnp.sum(et * et, axis=0, keepdims=True)
        e2c_ref[...] = jnp.transpose(e2)
        et2_ref[...] = et + et

    e2c = e2c_ref[...]
    et2 = et2_ref[...]
    iota_f = lax.broadcasted_iota(jnp.int32, (k, t), 0).astype(jnp.float32)
    for b in range(_B_BLK):
        xb = xt_ref[b]
        x2 = jnp.sum(xb * xb, axis=0, keepdims=True)
        xe2 = lax.dot_general(
            et2, xb, (((0,), (0,)), ((), ())),
            preferred_element_type=jnp.float32)
        val = (x2 - xe2) + e2c
        idx = iota_f
        while val.shape[0] > 1:
            h = val.shape[0] // 2
            keep = val[:h] <= val[h:]
            val = jnp.where(keep, val[:h], val[h:])
            idx = jnp.where(keep, idx[:h], idx[h:])
        ind3_ref[b] = idx.astype(jnp.int32)


def _argmin_indices(x_t, embed_t):
    b, d, t = x_t.shape
    k = embed_t.shape[1]
    ind3 = pl.pallas_call(
        _argmin_body,
        grid=(b // _B_BLK,),
        in_specs=[
            pl.BlockSpec((_B_BLK, d, t), lambda i: (i, 0, 0)),
            pl.BlockSpec((d, k), lambda i: (0, 0)),
        ],
        out_specs=pl.BlockSpec((_B_BLK, 1, t), lambda i: (i, 0, 0)),
        out_shape=jax.ShapeDtypeStruct((b, 1, t), jnp.int32),
        scratch_shapes=[
            pltpu.VMEM((k, 1), jnp.float32),
            pltpu.VMEM((d, k), jnp.float32),
        ],
    )(x_t, embed_t)
    return ind3


@functools.cache
def _make_sc_gather(K, D, B, T):
    info = plsc.get_sparse_core_info()
    NC, NS, L = info.num_cores, info.num_subcores, info.num_lanes
    NW = NC * NS
    BG, DG = 8, 4
    BPW, DPW = B // BG, D // DG
    assert BG * DG == NW and T % L == 0
    mesh = plsc.VectorSubcoreMesh(core_axis_name="c", subcore_axis_name="s")

    @functools.partial(
        pl.kernel,
        mesh=mesh,
        out_type=jax.ShapeDtypeStruct((B, D, T), jnp.float32),
        scratch_types=[
            pltpu.VMEM((BPW, 1, T), jnp.int32),
            pltpu.VMEM((K * DPW,), jnp.float32),
            pltpu.VMEM((BPW, DPW, T), jnp.float32),
        ],
        compiler_params=pltpu.CompilerParams(needs_layout_passes=False),
    )
    def gather(et_flat_hbm, idx_hbm, out_hbm, idx_v, tab_v, out_v):
        wid = lax.axis_index("s") * NC + lax.axis_index("c")
        bg = wid // DG
        dg = wid % DG
        pltpu.sync_copy(idx_hbm.at[pl.ds(bg * BPW, BPW)], idx_v)
        pltpu.sync_copy(et_flat_hbm.at[pl.ds(dg * (DPW * K), DPW * K)], tab_v)

        for b in range(BPW):
            def body(g, _, b=b):
                col = g * L
                idx16 = idx_v[b, 0, pl.ds(col, L)]
                vals = [plsc.load_gather(tab_v, [idx16 + (d * K)])
                        for d in range(DPW)]
                for d in range(DPW):
                    out_v[b, d, pl.ds(col, L)] = vals[d]
                return _

            lax.fori_loop(0, T // L, body, None, unroll=4)

        pltpu.sync_copy(
            out_v, out_hbm.at[pl.ds(bg * BPW, BPW), pl.ds(dg * DPW, DPW)])

    return gather


def kernel(x, embed):
    b, t, d = x.shape
    k = embed.shape[0]
    x_t = x.transpose(0, 2, 1)
    embed_t = embed.T
    ind3 = _argmin_indices(x_t, embed_t)
    quant_t = _make_sc_gather(k, d, b, t)(embed_t.reshape(k * d), ind3)
    return quant_t.transpose(0, 2, 1), ind3.reshape(b, t)


# speedup vs baseline: 1.0800x; 1.0618x over previous
import functools

import jax
import jax.numpy as jnp
from jax import lax
from jax.experimental import pallas as pl
from jax.experimental.pallas import tpu as pltpu
from jax.experimental.pallas import tpu_sc as plsc


_B_BLK = 8


def _argmin_body(xt_ref, et_ref, ind3_ref, e2c_ref, et2_ref):
    t = xt_ref.shape[2]
    k = et_ref.shape[1]

    @pl.when(pl.program_id(0) == 0)
    def _():
        et = et_ref[...]
        e2 = jnp.sum(et * et, axis=0, keepdims=True)
        e2c_ref[...] = jnp.transpose(e2)
        et2_ref[...] = et + et

    e2c = e2c_ref[...]
    et2 = et2_ref[...]
    iota_f = lax.broadcasted_iota(jnp.int32, (k, t), 0).astype(jnp.float32)
    for b in range(_B_BLK):
        xb = xt_ref[b]
        x2 = jnp.sum(xb * xb, axis=0, keepdims=True)
        xe2 = lax.dot_general(
            et2, xb, (((0,), (0,)), ((), ())),
            preferred_element_type=jnp.float32)
        val = (x2 - xe2) + e2c
        idx = iota_f
        while val.shape[0] > 1:
            h = val.shape[0] // 2
            keep = val[:h] <= val[h:]
            val = jnp.where(keep, val[:h], val[h:])
            idx = jnp.where(keep, idx[:h], idx[h:])
        ind3_ref[b] = idx.astype(jnp.int32)


def _argmin_indices(x_t, embed_t):
    b, d, t = x_t.shape
    k = embed_t.shape[1]
    ind3 = pl.pallas_call(
        _argmin_body,
        grid=(b // _B_BLK,),
        in_specs=[
            pl.BlockSpec((_B_BLK, d, t), lambda i: (i, 0, 0)),
            pl.BlockSpec((d, k), lambda i: (0, 0)),
        ],
        out_specs=pl.BlockSpec((_B_BLK, 1, t), lambda i: (i, 0, 0)),
        out_shape=jax.ShapeDtypeStruct((b, 1, t), jnp.int32),
        scratch_shapes=[
            pltpu.VMEM((k, 1), jnp.float32),
            pltpu.VMEM((d, k), jnp.float32),
        ],
    )(x_t, embed_t)
    return ind3


@functools.cache
def _make_sc_gather(K, D, B, T):
    info = plsc.get_sparse_core_info()
    NC, NS, L = info.num_cores, info.num_subcores, info.num_lanes
    NW = NC * NS
    BG, DG = 8, 4
    BPW, DPW = B // BG, D // DG
    assert BG * DG == NW and T % L == 0
    mesh = plsc.VectorSubcoreMesh(core_axis_name="c", subcore_axis_name="s")

    @functools.partial(
        pl.kernel,
        mesh=mesh,
        out_type=jax.ShapeDtypeStruct((B, D, T), jnp.float32),
        scratch_types=[
            pltpu.VMEM((BPW, 1, T), jnp.int32),
            pltpu.VMEM((K * DPW,), jnp.float32),
            pltpu.VMEM((BPW, DPW, T), jnp.float32),
        ],
        compiler_params=pltpu.CompilerParams(needs_layout_passes=False),
    )
    def gather(et_flat_hbm, idx_hbm, out_hbm, idx_v, tab_v, out_v):
        wid = lax.axis_index("s") * NC + lax.axis_index("c")
        bg = wid // DG
        dg = wid % DG
        pltpu.sync_copy(idx_hbm.at[pl.ds(bg * BPW, BPW)], idx_v)
        pltpu.sync_copy(et_flat_hbm.at[pl.ds(dg * (DPW * K), DPW * K)], tab_v)

        for b in range(BPW):
            def body(g, _, b=b):
                col = g * L
                idx16 = idx_v[b, 0, pl.ds(col, L)]
                vals = [plsc.load_gather(tab_v, [idx16 + (d * K)])
                        for d in range(DPW)]
                for d in range(DPW):
                    out_v[b, d, pl.ds(col, L)] = vals[d]
                return _

            lax.fori_loop(0, T // L, body, None, unroll=4)

        pltpu.sync_copy(
            out_v, out_hbm.at[pl.ds(bg * BPW, BPW), pl.ds(dg * DPW, DPW)])

    return gather


def kernel(x, embed):
    b, t, d = x.shape
    k = embed.shape[0]
    x_t = x.transpose(0, 2, 1)
    embed_t = embed.T
    ind3 = _argmin_indices(x_t, embed_t)
    quant_t = _make_sc_gather(k, d, b, t)(embed_t.reshape(k * d), ind3)
    return quant_t.transpose(0, 2, 1), ind3.reshape(b, t)


# speedup vs baseline: 1.0939x; 1.0129x over previous
import functools

import jax
import jax.numpy as jnp
from jax import lax
from jax.experimental import pallas as pl
from jax.experimental.pallas import tpu as pltpu
from jax.experimental.pallas import tpu_sc as plsc


_B_BLK = 16


def _argmin_body(xt_ref, et_ref, ind3_ref, e2c_ref, et2_ref):
    t = xt_ref.shape[2]
    k = et_ref.shape[1]

    @pl.when(pl.program_id(0) == 0)
    def _():
        et = et_ref[...]
        e2 = jnp.sum(et * et, axis=0, keepdims=True)
        e2c_ref[...] = jnp.transpose(e2)
        et2_ref[...] = et + et

    e2c = e2c_ref[...]
    et2 = et2_ref[...]
    iota_f = lax.broadcasted_iota(jnp.int32, (k, t), 0).astype(jnp.float32)
    for b in range(_B_BLK):
        xb = xt_ref[b]
        x2 = jnp.sum(xb * xb, axis=0, keepdims=True)
        xe2 = lax.dot_general(
            et2, xb, (((0,), (0,)), ((), ())),
            preferred_element_type=jnp.float32)
        val = (x2 - xe2) + e2c
        idx = iota_f
        while val.shape[0] > 1:
            h = val.shape[0] // 2
            keep = val[:h] <= val[h:]
            val = jnp.where(keep, val[:h], val[h:])
            idx = jnp.where(keep, idx[:h], idx[h:])
        ind3_ref[b] = idx.astype(jnp.int32)


def _argmin_indices(x_t, embed_t):
    b, d, t = x_t.shape
    k = embed_t.shape[1]
    ind3 = pl.pallas_call(
        _argmin_body,
        grid=(b // _B_BLK,),
        in_specs=[
            pl.BlockSpec((_B_BLK, d, t), lambda i: (i, 0, 0)),
            pl.BlockSpec((d, k), lambda i: (0, 0)),
        ],
        out_specs=pl.BlockSpec((_B_BLK, 1, t), lambda i: (i, 0, 0)),
        out_shape=jax.ShapeDtypeStruct((b, 1, t), jnp.int32),
        scratch_shapes=[
            pltpu.VMEM((k, 1), jnp.float32),
            pltpu.VMEM((d, k), jnp.float32),
        ],
    )(x_t, embed_t)
    return ind3


@functools.cache
def _make_sc_gather(K, D, B, T):
    info = plsc.get_sparse_core_info()
    NC, NS, L = info.num_cores, info.num_subcores, info.num_lanes
    NW = NC * NS
    BG, DG = 8, 4
    BPW, DPW = B // BG, D // DG
    assert BG * DG == NW and T % L == 0
    mesh = plsc.VectorSubcoreMesh(core_axis_name="c", subcore_axis_name="s")

    @functools.partial(
        pl.kernel,
        mesh=mesh,
        out_type=jax.ShapeDtypeStruct((B, D, T), jnp.float32),
        scratch_types=[
            pltpu.VMEM((BPW, 1, T), jnp.int32),
            pltpu.VMEM((K * DPW,), jnp.float32),
            pltpu.VMEM((BPW, DPW, T), jnp.float32),
        ],
        compiler_params=pltpu.CompilerParams(needs_layout_passes=False),
    )
    def gather(et_flat_hbm, idx_hbm, out_hbm, idx_v, tab_v, out_v):
        wid = lax.axis_index("s") * NC + lax.axis_index("c")
        bg = wid // DG
        dg = wid % DG
        pltpu.sync_copy(idx_hbm.at[pl.ds(bg * BPW, BPW)], idx_v)
        pltpu.sync_copy(et_flat_hbm.at[pl.ds(dg * (DPW * K), DPW * K)], tab_v)

        for b in range(BPW):
            def body(g, _, b=b):
                col = g * L
                idx16 = idx_v[b, 0, pl.ds(col, L)]
                vals = [plsc.load_gather(tab_v, [idx16 + (d * K)])
                        for d in range(DPW)]
                for d in range(DPW):
                    out_v[b, d, pl.ds(col, L)] = vals[d]
                return _

            lax.fori_loop(0, T // L, body, None, unroll=4)

        pltpu.sync_copy(
            out_v, out_hbm.at[pl.ds(bg * BPW, BPW), pl.ds(dg * DPW, DPW)])

    return gather


def kernel(x, embed):
    b, t, d = x.shape
    k = embed.shape[0]
    x_t = x.transpose(0, 2, 1)
    embed_t = embed.T
    ind3 = _argmin_indices(x_t, embed_t)
    quant_t = _make_sc_gather(k, d, b, t)(embed_t.reshape(k * d), ind3)
    return quant_t.transpose(0, 2, 1), ind3.reshape(b, t)
